# baseline (speedup 1.0000x reference)
import jax
import jax.numpy as jnp
from jax.experimental import pallas as pl
from jax.experimental.pallas import tpu as pltpu

_ALPHA = 0.25
_B, _N, _M, _C = 4, 50000, 32, 80
_T = 10000
_NT = _N // _T

_DN = (((1,), (0,)), ((), ()))


def _loss_body(cls_hbm, bbox_ref, anc_ref, boxes_ref, boxes_t_ref,
               labels_ref, out_ref, acc, buf, sem):
    b = pl.program_id(0)
    j = pl.program_id(1)
    step = b * _NT + j
    p = jax.lax.rem(step, 2)
    np_ = 1 - p

    def start_copy(slot, bb, jj):
        pltpu.make_async_copy(
            cls_hbm.at[bb, pl.ds(jj * _T, _T), :],
            buf.at[slot],
            sem.at[slot],
        ).start()

    @pl.when(step == 0)
    def _():
        out_ref[0] = 0.0
        out_ref[1] = 0.0
        start_copy(0, 0, 0)

    @pl.when(j == 0)
    def _():
        acc[0] = 0.0
        acc[1] = 0.0
        acc[2] = 0.0

    @pl.when(step < _B * _NT - 1)
    def _():
        nstep = step + 1
        nb = nstep // _NT
        nj = jax.lax.rem(nstep, _NT)
        start_copy(np_, nb, nj)

    a = anc_ref[...]
    acx = a[0:1, :]
    acy = a[1:2, :]
    aw = a[2:3, :]
    ah = a[3:4, :]
    al = acx - aw * 0.5
    at_ = acy - ah * 0.5
    ar = acx + aw * 0.5
    ab = acy + ah * 0.5

    bx = boxes_ref[...]
    bcx = bx[:, 0:1]
    bcy = bx[:, 1:2]
    bw = bx[:, 2:3]
    bh = bx[:, 3:4]
    bl = bcx - bw * 0.5
    bt = bcy - bh * 0.5
    br = bcx + bw * 0.5
    bb_ = bcy + bh * 0.5

    ix0 = jnp.maximum(al, bl)
    iy0 = jnp.maximum(at_, bt)
    ix1 = jnp.minimum(ar, br)
    iy1 = jnp.minimum(ab, bb_)
    iw = jnp.clip(ix1 - ix0, 0.0, None)
    ih = jnp.clip(iy1 - iy0, 0.0, None)
    inter = iw * ih
    area_a = (ar - al) * (ab - at_)
    area_b = (br - bl) * (bb_ - bt)
    union = area_a + area_b - inter
    iou = inter / union

    vals = jnp.max(iou, axis=0, keepdims=True)
    ids32 = jax.lax.broadcasted_iota(jnp.int32, (_M, _T), 0)
    idx = jnp.min(jnp.where(iou == vals, ids32, _M), axis=0, keepdims=True)

    fg = vals > 0.5
    fg_f = fg.astype(jnp.float32)
    cm_row = (fg | (vals < 0.4)).astype(jnp.float32)

    safe2 = jnp.where(fg, idx, -1)
    oh32 = (ids32 == safe2).astype(jnp.float32)

    tgt4 = jax.lax.dot_general(boxes_t_ref[...], oh32, _DN,
                               preferred_element_type=jnp.float32)

    pr = bbox_ref[...]
    t_cx = (tgt4[0:1, :] - acx) / aw * 10.0
    t_cy = (tgt4[1:2, :] - acy) / ah * 10.0
    t_w = jnp.log(tgt4[2:3, :] / aw + 1e-8) * 5.0
    t_h = jnp.log(tgt4[3:4, :] / ah + 1e-8) * 5.0

    def sl1(d):
        return jnp.where(d < 1.0, 0.5 * d * d, d - 0.5)

    sl1_row = (sl1(jnp.abs(pr[0:1, :] - t_cx)) + sl1(jnp.abs(pr[1:2, :] - t_cy))
               + sl1(jnp.abs(pr[2:3, :] - t_w)) + sl1(jnp.abs(pr[3:4, :] - t_h)))
    sl1_tile = jnp.sum(sl1_row * fg_f)

    pltpu.make_async_copy(
        cls_hbm.at[b, pl.ds(j * _T, _T), :], buf.at[p], sem.at[p]
    ).wait()
    x = buf[p]
    u = jnp.exp(-jnp.abs(x))
    lg = jnp.log(1.0 + u)
    sp = jnp.maximum(x, 0.0) + lg
    t2 = x - sp
    l0 = sp * jnp.exp(t2 + t2)
    l1 = -t2 * jnp.exp(-(sp + sp))

    l0b = l0.astype(jnp.bfloat16)
    l1b = l1.astype(jnp.bfloat16)
    term1 = jnp.sum(jax.lax.dot_general(cm_row.astype(jnp.bfloat16), l0b, _DN,
                                        preferred_element_type=jnp.float32))
    oh32b = oh32.astype(jnp.bfloat16)
    p0 = jax.lax.dot_general(oh32b, l0b, _DN,
                             preferred_element_type=jnp.float32)
    p1 = jax.lax.dot_general(oh32b, l1b, _DN,
                             preferred_element_type=jnp.float32)
    labs = labels_ref[...]
    cls_ids = jax.lax.broadcasted_iota(jnp.int32, (_M, _C), 1)
    labmat = (labs == cls_ids.astype(jnp.float32) + 1.0).astype(jnp.float32)
    focal_tile = (0.25 * (term1 - jnp.sum(p0 * labmat))
                  + 0.75 * jnp.sum(p1 * labmat))

    acc[0] += jnp.sum(fg_f)
    acc[1] += sl1_tile
    acc[2] += focal_tile

    @pl.when(j == _NT - 1)
    def _():
        n_fg = acc[0]
        bb_loss = jnp.where(n_fg > 0.0,
                            acc[1] / jnp.maximum(n_fg * 4.0, 1.0), 0.0)
        cls_loss = acc[2] / jnp.maximum(n_fg, 1.0)
        out_ref[0] += cls_loss / _B
        out_ref[1] += bb_loss / _B


def kernel(cls_preds, bbox_preds, anchors, boxes, labels):
    bbox_t = jnp.transpose(bbox_preds.reshape(_B, _NT, _T, 4), (0, 1, 3, 2))
    anchors_t = jnp.transpose(anchors.reshape(_B, _NT, _T, 4), (0, 1, 3, 2))
    boxes_t = jnp.transpose(boxes, (0, 2, 1))
    labels_f = labels.astype(jnp.float32).reshape(_B, _M, 1)

    out = pl.pallas_call(
        _loss_body,
        grid=(_B, _NT),
        in_specs=[
            pl.BlockSpec(memory_space=pl.ANY),
            pl.BlockSpec((None, None, 4, _T), lambda b, j: (b, j, 0, 0)),
            pl.BlockSpec((None, None, 4, _T), lambda b, j: (b, j, 0, 0)),
            pl.BlockSpec((None, _M, 4), lambda b, j: (b, 0, 0)),
            pl.BlockSpec((None, 4, _M), lambda b, j: (b, 0, 0)),
            pl.BlockSpec((None, _M, 1), lambda b, j: (b, 0, 0)),
        ],
        out_specs=pl.BlockSpec(memory_space=pltpu.SMEM),
        out_shape=jax.ShapeDtypeStruct((2,), jnp.float32),
        scratch_shapes=[
            pltpu.SMEM((3,), jnp.float32),
            pltpu.VMEM((2, _T, _C), jnp.float32),
            pltpu.SemaphoreType.DMA((2,)),
        ],
    )(cls_preds, bbox_t, anchors_t, boxes, boxes_t, labels_f)
    return out


# speedup vs baseline: 1.1044x; 1.1044x over previous
import jax
import jax.numpy as jnp
from jax.experimental import pallas as pl
from jax.experimental.pallas import tpu as pltpu

_ALPHA = 0.25
_B, _N, _M, _C = 4, 50000, 32, 80
_T = 10000
_NT = _N // _T

_DN = (((1,), (0,)), ((), ()))


def _loss_body(cls_hbm, bbox_ref, anc_ref, boxes_ref, boxes_t_ref,
               labels_ref, out_ref, acc, buf, sem):
    b = pl.program_id(0)
    j = pl.program_id(1)
    step = b * _NT + j
    p = jax.lax.rem(step, 2)
    np_ = 1 - p

    def start_copy(slot, bb, jj):
        pltpu.make_async_copy(
            cls_hbm.at[bb, pl.ds(jj * _T, _T), :],
            buf.at[slot],
            sem.at[slot],
        ).start()

    @pl.when(step == 0)
    def _():
        out_ref[0] = 0.0
        out_ref[1] = 0.0
        start_copy(0, 0, 0)

    @pl.when(j == 0)
    def _():
        acc[0] = 0.0
        acc[1] = 0.0
        acc[2] = 0.0

    @pl.when(step < _B * _NT - 1)
    def _():
        nstep = step + 1
        nb = nstep // _NT
        nj = jax.lax.rem(nstep, _NT)
        start_copy(np_, nb, nj)

    a = anc_ref[...]
    acx = a[0:1, :]
    acy = a[1:2, :]
    aw = a[2:3, :]
    ah = a[3:4, :]
    al = acx - aw * 0.5
    at_ = acy - ah * 0.5
    ar = acx + aw * 0.5
    ab = acy + ah * 0.5

    bx = boxes_ref[...]
    bcx = bx[:, 0:1]
    bcy = bx[:, 1:2]
    bw = bx[:, 2:3]
    bh = bx[:, 3:4]
    bl = bcx - bw * 0.5
    bt = bcy - bh * 0.5
    br = bcx + bw * 0.5
    bb_ = bcy + bh * 0.5

    ix0 = jnp.maximum(al, bl)
    iy0 = jnp.maximum(at_, bt)
    ix1 = jnp.minimum(ar, br)
    iy1 = jnp.minimum(ab, bb_)
    iw = jnp.clip(ix1 - ix0, 0.0, None)
    ih = jnp.clip(iy1 - iy0, 0.0, None)
    inter = iw * ih
    area_a = (ar - al) * (ab - at_)
    area_b = (br - bl) * (bb_ - bt)
    union = area_a + area_b - inter
    iou = inter / union

    vals = jnp.max(iou, axis=0, keepdims=True)
    ids32 = jax.lax.broadcasted_iota(jnp.int32, (_M, _T), 0)
    idx = jnp.min(jnp.where(iou == vals, ids32, _M), axis=0, keepdims=True)

    fg = vals > 0.5
    fg_f = fg.astype(jnp.float32)
    cm_row = (fg | (vals < 0.4)).astype(jnp.float32)

    safe2 = jnp.where(fg, idx, -1)
    oh32 = (ids32 == safe2).astype(jnp.float32)

    tgt4 = jax.lax.dot_general(boxes_t_ref[...], oh32, _DN,
                               preferred_element_type=jnp.float32)

    pr = bbox_ref[...]
    t_cx = (tgt4[0:1, :] - acx) / aw * 10.0
    t_cy = (tgt4[1:2, :] - acy) / ah * 10.0
    t_w = jnp.log(tgt4[2:3, :] / aw + 1e-8) * 5.0
    t_h = jnp.log(tgt4[3:4, :] / ah + 1e-8) * 5.0

    def sl1(d):
        return jnp.where(d < 1.0, 0.5 * d * d, d - 0.5)

    sl1_row = (sl1(jnp.abs(pr[0:1, :] - t_cx)) + sl1(jnp.abs(pr[1:2, :] - t_cy))
               + sl1(jnp.abs(pr[2:3, :] - t_w)) + sl1(jnp.abs(pr[3:4, :] - t_h)))
    sl1_tile = jnp.sum(sl1_row * fg_f)

    pltpu.make_async_copy(
        cls_hbm.at[b, pl.ds(j * _T, _T), :], buf.at[p], sem.at[p]
    ).wait()
    x = buf[p].astype(jnp.bfloat16)
    u = jnp.exp(-jnp.abs(x))
    lg = jnp.log(1.0 + u)
    sp = jnp.maximum(x, 0.0) + lg
    r = 1.0 / (1.0 + u)
    r2 = r * r
    u2r2 = (u * u) * r2
    pos = x >= 0.0
    s2 = jnp.where(pos, r2, u2r2)
    oms2 = jnp.where(pos, u2r2, r2)
    l0 = sp * s2
    l1 = (sp - x) * oms2

    term1 = jnp.sum(jax.lax.dot_general(cm_row.astype(jnp.bfloat16), l0, _DN,
                                        preferred_element_type=jnp.float32))
    oh32b = oh32.astype(jnp.bfloat16)
    p0 = jax.lax.dot_general(oh32b, l0, _DN,
                             preferred_element_type=jnp.float32)
    p1 = jax.lax.dot_general(oh32b, l1, _DN,
                             preferred_element_type=jnp.float32)
    labs = labels_ref[...]
    cls_ids = jax.lax.broadcasted_iota(jnp.int32, (_M, _C), 1)
    labmat = (labs == cls_ids.astype(jnp.float32) + 1.0).astype(jnp.float32)
    focal_tile = (0.25 * (term1 - jnp.sum(p0 * labmat))
                  + 0.75 * jnp.sum(p1 * labmat))

    acc[0] += jnp.sum(fg_f)
    acc[1] += sl1_tile
    acc[2] += focal_tile

    @pl.when(j == _NT - 1)
    def _():
        n_fg = acc[0]
        bb_loss = jnp.where(n_fg > 0.0,
                            acc[1] / jnp.maximum(n_fg * 4.0, 1.0), 0.0)
        cls_loss = acc[2] / jnp.maximum(n_fg, 1.0)
        out_ref[0] += cls_loss / _B
        out_ref[1] += bb_loss / _B


def kernel(cls_preds, bbox_preds, anchors, boxes, labels):
    bbox_t = jnp.transpose(bbox_preds.reshape(_B, _NT, _T, 4), (0, 1, 3, 2))
    anchors_t = jnp.transpose(anchors.reshape(_B, _NT, _T, 4), (0, 1, 3, 2))
    boxes_t = jnp.transpose(boxes, (0, 2, 1))
    labels_f = labels.astype(jnp.float32).reshape(_B, _M, 1)

    out = pl.pallas_call(
        _loss_body,
        grid=(_B, _NT),
        in_specs=[
            pl.BlockSpec(memory_space=pl.ANY),
            pl.BlockSpec((None, None, 4, _T), lambda b, j: (b, j, 0, 0)),
            pl.BlockSpec((None, None, 4, _T), lambda b, j: (b, j, 0, 0)),
            pl.BlockSpec((None, _M, 4), lambda b, j: (b, 0, 0)),
            pl.BlockSpec((None, 4, _M), lambda b, j: (b, 0, 0)),
            pl.BlockSpec((None, _M, 1), lambda b, j: (b, 0, 0)),
        ],
        out_specs=pl.BlockSpec(memory_space=pltpu.SMEM),
        out_shape=jax.ShapeDtypeStruct((2,), jnp.float32),
        scratch_shapes=[
            pltpu.SMEM((3,), jnp.float32),
            pltpu.VMEM((2, _T, _C), jnp.float32),
            pltpu.SemaphoreType.DMA((2,)),
        ],
    )(cls_preds, bbox_t, anchors_t, boxes, boxes_t, labels_f)
    return out
